# Initial kernel scaffold; baseline (speedup 1.0000x reference)
import jax
import jax.numpy as jnp
from jax.experimental import pallas as pl


def kernel(nodes, edges, senders, receivers, W1, b1, W2, b2, W3, b3, W4, b4, W5, b5):
    raise NotImplementedError("write your pallas kernel here")


# speedup vs baseline: 8.2991x; 8.2991x over previous
import functools

import jax
import jax.numpy as jnp
from jax import lax
from jax.experimental import pallas as pl
from jax.experimental.pallas import tpu as pltpu
from jax.experimental.pallas import tpu_sc as plsc

N, E, DF, DE, DO = 10000, 320000, 128, 16, 128
NC, NS = 2, 16
NW = NC * NS
EPW = E // NW
BK = 80
NB = EPW // BK
DC = DO + 16
RPT = N // NS
RC = 125


def _node_dense_body(x_ref, w1_ref, b1_ref, w2_ref, b2_ref, w4_ref, w5_ref,
                     b5_ref, A_ref, B_ref, C_ref, a_ref, bv_ref):
    x = x_ref[...]
    A = jnp.dot(x, w1_ref[...], preferred_element_type=jnp.float32) + b1_ref[...]
    B = jnp.dot(x, w2_ref[...], preferred_element_type=jnp.float32) + b2_ref[...]
    C = jnp.dot(x, w5_ref[...], preferred_element_type=jnp.float32) + b5_ref[...]
    A_ref[...] = A
    B_ref[...] = B
    C_ref[...] = C
    w4 = w4_ref[...]
    a_ref[...] = jnp.dot(A, w4, preferred_element_type=jnp.float32)
    bv_ref[...] = jnp.dot(B, w4, preferred_element_type=jnp.float32)


def _edge_dense_body(e_ref, w3_ref, b3_ref, w4_ref, b4_ref, EF0_ref, e0_ref):
    ew = e_ref[...]
    EF0 = jnp.dot(ew, w3_ref[...], preferred_element_type=jnp.float32) + b3_ref[...]
    EF0_ref[...] = EF0
    w34 = jnp.dot(w3_ref[...], w4_ref[...], preferred_element_type=jnp.float32)
    c34 = jnp.dot(b3_ref[...], w4_ref[...], preferred_element_type=jnp.float32)
    e0_ref[...] = (jnp.dot(ew, w34, preferred_element_type=jnp.float32)
                   + c34 + b4_ref[...])


def _combine_body(p0_ref, p1_ref, out_ref):
    s = p0_ref[...] + p1_ref[...]
    den = s[:, DO:DO + 1]
    num = s[:, :DO]
    out_ref[...] = jnp.where(den > 0.0, num / den, 0.0)


def _sc_body(s3, r3, e3, a_hbm, b_hbm, A_hbm, B_hbm, Cw_hbm, EF0_hbm,
             ef_out, acc_out,
             s_v, r_v, e_v, ex_v, ag_v, bg_v, bufE, bufC,
             acc_sh, semE, semG, semC, semS):
    cid = lax.axis_index("c")
    sid = lax.axis_index("s")
    wid = cid * NS + sid

    def _zrow(i, _):
        for q in range(DC // 16):
            bufC[i, pl.ds(q * 16, 16)] = jnp.zeros((16,), jnp.float32)
        return 0
    lax.fori_loop(0, BK, _zrow, 0)
    for t in range(RPT // BK):
        pltpu.sync_copy(bufC, acc_sh.at[pl.ds(sid * RPT + t * BK, BK)])
    rem = RPT - (RPT // BK) * BK
    if rem:
        pltpu.sync_copy(bufC.at[pl.ds(0, rem)],
                        acc_sh.at[pl.ds(sid * RPT + (RPT // BK) * BK, rem)])
    plsc.subcore_barrier()

    def _batch(b, _):
        cpS = pltpu.async_copy(s3.at[wid, b], s_v, semS)
        cpR = pltpu.async_copy(r3.at[wid, b], r_v, semS)
        cpE0 = pltpu.async_copy(e3.at[wid, b], e_v, semS)
        cpS.wait()
        cpR.wait()
        cpa = pltpu.async_copy(a_hbm.at[s_v], ag_v, semG)
        cpb = pltpu.async_copy(b_hbm.at[r_v], bg_v, semG)
        gbase = wid * EPW + b * BK
        cpEF = pltpu.async_copy(EF0_hbm.at[pl.ds(gbase, BK)], bufE, semE)
        cpC = pltpu.async_copy(Cw_hbm.at[s_v], bufC, semC)
        cpE0.wait()
        cpa.wait()
        cpb.wait()

        for q in range(BK // 16):
            sl = pl.ds(q * 16, 16)
            att = ag_v[sl] + bg_v[sl] + e_v[sl]
            att = jnp.where(att >= 0.0, att, 0.01 * att)
            ex_v[sl] = jnp.exp(att)

        cpEF.wait()
        cpA = pltpu.async_copy(A_hbm.at[s_v], bufE, semE, add=True)
        cpB = pltpu.async_copy(B_hbm.at[r_v], bufE, semE, add=True)
        cpC.wait()

        def _scale(e, _):
            exb = plsc.load_gather(ex_v, [jnp.full((16,), e, jnp.int32)])
            for q in range(DC // 16):
                sl = pl.ds(q * 16, 16)
                bufC[e, sl] = bufC[e, sl] * exb
            return 0
        lax.fori_loop(0, BK, _scale, 0)

        cpA.wait()
        cpB.wait()
        pltpu.sync_copy(bufE, ef_out.at[pl.ds(gbase, BK)])
        pltpu.sync_copy(bufC, acc_sh.at[r_v], add=True)
        return 0

    lax.fori_loop(0, NB, _batch, 0)
    plsc.subcore_barrier()

    pltpu.sync_copy(acc_sh.at[pl.ds(sid * RPT, RPT)],
                    acc_out.at[pl.ds(cid * N + sid * RPT, RPT)])


def _node_dense(nodes, W1, b1, W2, b2, W4, W5, b5):
    blk = 2000
    grid = (N // blk,)
    full = lambda shape: pl.BlockSpec(shape, lambda i: (0, 0))
    return pl.pallas_call(
        _node_dense_body,
        grid=grid,
        in_specs=[
            pl.BlockSpec((blk, DF), lambda i: (i, 0)),
            full((DF, DO)), full((1, DO)),
            full((DF, DO)), full((1, DO)),
            full((DO, 1)),
            full((DF, DO)), full((1, DO)),
        ],
        out_specs=[
            pl.BlockSpec((blk, DO), lambda i: (i, 0)),
            pl.BlockSpec((blk, DO), lambda i: (i, 0)),
            pl.BlockSpec((blk, DO), lambda i: (i, 0)),
            pl.BlockSpec((blk, 1), lambda i: (i, 0)),
            pl.BlockSpec((blk, 1), lambda i: (i, 0)),
        ],
        out_shape=[
            jax.ShapeDtypeStruct((N, DO), jnp.float32),
            jax.ShapeDtypeStruct((N, DO), jnp.float32),
            jax.ShapeDtypeStruct((N, DO), jnp.float32),
            jax.ShapeDtypeStruct((N, 1), jnp.float32),
            jax.ShapeDtypeStruct((N, 1), jnp.float32),
        ],
    )(nodes, W1, b1.reshape(1, DO), W2, b2.reshape(1, DO), W4, W5,
      b5.reshape(1, DO))


def _edge_dense(edges, W3, b3, W4, b4):
    blk = 3200
    grid = (E // blk,)
    full = lambda shape: pl.BlockSpec(shape, lambda i: (0, 0))
    return pl.pallas_call(
        _edge_dense_body,
        grid=grid,
        in_specs=[
            pl.BlockSpec((blk, DE), lambda i: (i, 0)),
            full((DE, DO)), full((1, DO)),
            full((DO, 1)), full((1, 1)),
        ],
        out_specs=[
            pl.BlockSpec((blk, DO), lambda i: (i, 0)),
            pl.BlockSpec((blk, 1), lambda i: (i, 0)),
        ],
        out_shape=[
            jax.ShapeDtypeStruct((E, DO), jnp.float32),
            jax.ShapeDtypeStruct((E, 1), jnp.float32),
        ],
    )(edges, W3, b3.reshape(1, DO), W4, b4.reshape(1, 1))


def _combine(acc_flat):
    blk = 2000
    grid = (N // blk,)
    return pl.pallas_call(
        _combine_body,
        grid=grid,
        in_specs=[
            pl.BlockSpec((blk, DC), lambda i: (i, 0)),
            pl.BlockSpec((blk, DC), lambda i: (i + N // blk, 0)),
        ],
        out_specs=pl.BlockSpec((blk, DO), lambda i: (i, 0)),
        out_shape=jax.ShapeDtypeStruct((N, DO), jnp.float32),
    )(acc_flat, acc_flat)


@functools.cache
def _get_sc_edges():
    return pl.kernel(
        _sc_body,
        out_type=[
            jax.ShapeDtypeStruct((E, DO), jnp.float32),
            jax.ShapeDtypeStruct((NC * N, DC), jnp.float32),
        ],
        mesh=plsc.VectorSubcoreMesh(core_axis_name="c", subcore_axis_name="s"),
        scratch_types=[
            pltpu.VMEM((BK,), jnp.int32),
            pltpu.VMEM((BK,), jnp.int32),
            pltpu.VMEM((BK,), jnp.float32),
            pltpu.VMEM((BK,), jnp.float32),
            pltpu.VMEM((BK,), jnp.float32),
            pltpu.VMEM((BK,), jnp.float32),
            pltpu.VMEM((BK, DO), jnp.float32),
            pltpu.VMEM((BK, DC), jnp.float32),
            pltpu.VMEM_SHARED((N, DC), jnp.float32),
            pltpu.SemaphoreType.DMA,
            pltpu.SemaphoreType.DMA,
            pltpu.SemaphoreType.DMA,
            pltpu.SemaphoreType.DMA,
        ],
        compiler_params=pltpu.CompilerParams(use_tc_tiling_on_sc=False,
                                             needs_layout_passes=False),
    )


def kernel(nodes, edges, senders, receivers, W1, b1, W2, b2, W3, b3, W4, b4,
           W5, b5):
    A, B, C, a, bv = _node_dense(nodes, W1, b1, W2, b2, W4, W5, b5)
    EF0, e0 = _edge_dense(edges, W3, b3, W4, b4)

    ones_col = jnp.concatenate(
        [jnp.ones((N, 1), jnp.float32), jnp.zeros((N, DC - DO - 1), jnp.float32)],
        axis=1)
    Cw = jnp.concatenate([C, ones_col], axis=1)

    s3 = senders.reshape(NW, NB, BK)
    r3 = receivers.reshape(NW, NB, BK)
    e3 = e0.reshape(NW, NB, BK)

    ef, acc_flat = _get_sc_edges()(s3, r3, e3, a.reshape(N), bv.reshape(N),
                                   A, B, Cw, EF0)
    new_nodes = _combine(acc_flat)
    return new_nodes, ef
